# baseline (speedup 1.0000x reference)
import functools

import jax
import jax.numpy as jnp
from jax import lax
from jax.experimental import pallas as pl
from jax.experimental.pallas import tpu as pltpu
from jax.experimental.pallas import tpu_sc as plsc

N_TOK = 8192
N_E = 8192
E_DIM = 256

TT = 1024
TCODE = 4096
NI = N_TOK // TT
NJ = N_E // TCODE

SC_NC = 2
SC_NS = 16
SC_NW = SC_NC * SC_NS
B_PER_W = N_TOK // SC_NW


NJ_HALF = NJ // 2


def _dist_body(zf_ref, szf_ref, se2_ref, emb_ref, idx_ref, loss_ref,
               min1, arg1, min2, arg2, acc):
    i = pl.program_id(0)
    j = pl.program_id(1)
    mm2 = lax.dot_general(zf_ref[...], emb_ref[...],
                          (((1,), (1,)), ((), ())),
                          preferred_element_type=jnp.float32)
    d = (szf_ref[...] + se2_ref[...]) + mm2
    lmin = jnp.min(d, axis=1, keepdims=True)
    iot = lax.broadcasted_iota(jnp.int32, d.shape, 1)
    larg = jnp.min(jnp.where(d == lmin, iot, jnp.int32(2 ** 30)),
                   axis=1, keepdims=True) + j * TCODE

    @pl.when(j == 0)
    def _():
        min1[...] = lmin
        arg1[...] = larg

    @pl.when((j > 0) & (j < NJ_HALF))
    def _():
        better = lmin < min1[...]
        min1[...] = jnp.where(better, lmin, min1[...])
        arg1[...] = jnp.where(better, larg, arg1[...])

    @pl.when(j == NJ_HALF)
    def _():
        min2[...] = lmin
        arg2[...] = larg

    @pl.when(j > NJ_HALF)
    def _():
        better = lmin < min2[...]
        min2[...] = jnp.where(better, lmin, min2[...])
        arg2[...] = jnp.where(better, larg, arg2[...])

    @pl.when(j == NJ - 1)
    def _():
        v1b = min1[...].astype(jnp.bfloat16).astype(jnp.float32)
        take2 = min2[...] < v1b
        idx_ref[...] = jnp.where(take2, arg2[...], arg1[...])
        dsel = jnp.where(take2, min2[...], min1[...])

        @pl.when(i == 0)
        def _():
            acc[0, 0] = 0.0

        acc[0, 0] += jnp.sum(dsel)

        @pl.when(i == NI - 1)
        def _():
            loss_ref[0, 0] = acc[0, 0]


def _dist_argmin(zf, szf, se2r, emb):
    return pl.pallas_call(
        _dist_body,
        grid=(NI, NJ),
        in_specs=[
            pl.BlockSpec((TT, E_DIM), lambda i, j: (i, 0)),
            pl.BlockSpec((TT, 1), lambda i, j: (i, 0)),
            pl.BlockSpec((1, TCODE), lambda i, j: (0, j)),
            pl.BlockSpec((TCODE, E_DIM), lambda i, j: (j, 0)),
        ],
        out_specs=[
            pl.BlockSpec((TT, 1), lambda i, j: (i, 0)),
            pl.BlockSpec((1, 1), lambda i, j: (0, 0),
                         memory_space=pltpu.SMEM),
        ],
        out_shape=[
            jax.ShapeDtypeStruct((N_TOK, 1), jnp.int32),
            jax.ShapeDtypeStruct((1, 1), jnp.float32),
        ],
        scratch_shapes=[
            pltpu.VMEM((TT, 1), jnp.float32),
            pltpu.VMEM((TT, 1), jnp.int32),
            pltpu.VMEM((TT, 1), jnp.float32),
            pltpu.VMEM((TT, 1), jnp.int32),
            pltpu.SMEM((1, 1), jnp.float32),
        ],
    )(zf, szf, se2r, emb)


def _make_gather():
    mesh = plsc.VectorSubcoreMesh(core_axis_name="c", subcore_axis_name="s")

    @functools.partial(
        pl.kernel, mesh=mesh,
        out_type=jax.ShapeDtypeStruct((N_TOK, E_DIM), jnp.float32),
        scratch_types=[
            pltpu.VMEM((B_PER_W,), jnp.int32),
            pltpu.VMEM((B_PER_W, E_DIM), jnp.float32),
            pltpu.SemaphoreType.DMA,
        ],
    )
    def _gather(table_hbm, idx_hbm, out_hbm, idx_v, rows_v, sem):
        wid = lax.axis_index("s") * SC_NC + lax.axis_index("c")
        base = wid * B_PER_W
        pltpu.sync_copy(idx_hbm.at[pl.ds(base, B_PER_W)], idx_v)
        pltpu.async_copy(table_hbm.at[idx_v], rows_v, sem).wait()
        pltpu.sync_copy(rows_v, out_hbm.at[pl.ds(base, B_PER_W)])

    return _gather


def kernel(z, emb):
    b, c, h, w = z.shape
    zp = jnp.transpose(z, (0, 2, 3, 1))
    zf = zp.reshape(-1, c)
    szf = jnp.sum(zf ** 2, axis=1, keepdims=True)
    se2 = jnp.sum(emb ** 2, axis=1)
    zfm2 = (zf * -2.0).astype(jnp.bfloat16)
    emb16 = emb.astype(jnp.bfloat16)
    idx2d, loss_sum = _dist_argmin(zfm2, szf, se2.reshape(1, N_E), emb16)
    zq_flat = _make_gather()(emb, idx2d.reshape(N_TOK))
    z_q = zq_flat.reshape(b, h, w, c).transpose(0, 3, 1, 2)
    codebook_loss = loss_sum[0, 0] / jnp.float32(N_TOK * E_DIM)
    cls_loss = jnp.float32(0.0)
    return (z_q, codebook_loss, cls_loss, idx2d)


# speedup vs baseline: 1.0395x; 1.0395x over previous
import functools

import jax
import jax.numpy as jnp
from jax import lax
from jax.experimental import pallas as pl
from jax.experimental.pallas import tpu as pltpu
from jax.experimental.pallas import tpu_sc as plsc

N_TOK = 8192
N_E = 8192
E_DIM = 256

TT = 1024
TCODE = 4096
NI = N_TOK // TT
NJ = N_E // TCODE

SC_NC = 2
SC_NS = 16
SC_NW = SC_NC * SC_NS
B_PER_W = N_TOK // SC_NW


NJ_HALF = NJ // 2


def _dist_body(zf_ref, szf_ref, se2_ref, emb_ref, idx_ref, loss_ref,
               min1, arg1, min2, arg2, acc):
    i = pl.program_id(0)
    j = pl.program_id(1)
    a16 = zf_ref[...].astype(jnp.bfloat16) * jnp.bfloat16(-2.0)
    mm2 = lax.dot_general(a16, emb_ref[...].astype(jnp.bfloat16),
                          (((1,), (1,)), ((), ())),
                          preferred_element_type=jnp.float32)
    d = (szf_ref[...] + se2_ref[...]) + mm2
    lmin = jnp.min(d, axis=1, keepdims=True)
    iot = lax.broadcasted_iota(jnp.int32, d.shape, 1)
    larg = jnp.min(jnp.where(d == lmin, iot, jnp.int32(2 ** 30)),
                   axis=1, keepdims=True) + j * TCODE

    @pl.when(j == 0)
    def _():
        min1[...] = lmin
        arg1[...] = larg

    @pl.when((j > 0) & (j < NJ_HALF))
    def _():
        better = lmin < min1[...]
        min1[...] = jnp.where(better, lmin, min1[...])
        arg1[...] = jnp.where(better, larg, arg1[...])

    @pl.when(j == NJ_HALF)
    def _():
        min2[...] = lmin
        arg2[...] = larg

    @pl.when(j > NJ_HALF)
    def _():
        better = lmin < min2[...]
        min2[...] = jnp.where(better, lmin, min2[...])
        arg2[...] = jnp.where(better, larg, arg2[...])

    @pl.when(j == NJ - 1)
    def _():
        v1b = min1[...].astype(jnp.bfloat16).astype(jnp.float32)
        take2 = min2[...] < v1b
        idx_ref[...] = jnp.where(take2, arg2[...], arg1[...])
        dsel = jnp.where(take2, min2[...], min1[...])

        @pl.when(i == 0)
        def _():
            acc[0, 0] = 0.0

        acc[0, 0] += jnp.sum(dsel)

        @pl.when(i == NI - 1)
        def _():
            loss_ref[0, 0] = acc[0, 0]


def _dist_argmin(zf, szf, se2r, emb):
    return pl.pallas_call(
        _dist_body,
        grid=(NI, NJ),
        in_specs=[
            pl.BlockSpec((TT, E_DIM), lambda i, j: (i, 0)),
            pl.BlockSpec((TT, 1), lambda i, j: (i, 0)),
            pl.BlockSpec((1, TCODE), lambda i, j: (0, j)),
            pl.BlockSpec((TCODE, E_DIM), lambda i, j: (j, 0)),
        ],
        out_specs=[
            pl.BlockSpec((TT, 1), lambda i, j: (i, 0)),
            pl.BlockSpec((1, 1), lambda i, j: (0, 0),
                         memory_space=pltpu.SMEM),
        ],
        out_shape=[
            jax.ShapeDtypeStruct((N_TOK, 1), jnp.int32),
            jax.ShapeDtypeStruct((1, 1), jnp.float32),
        ],
        scratch_shapes=[
            pltpu.VMEM((TT, 1), jnp.float32),
            pltpu.VMEM((TT, 1), jnp.int32),
            pltpu.VMEM((TT, 1), jnp.float32),
            pltpu.VMEM((TT, 1), jnp.int32),
            pltpu.SMEM((1, 1), jnp.float32),
        ],
    )(zf, szf, se2r, emb)


def _make_gather():
    mesh = plsc.VectorSubcoreMesh(core_axis_name="c", subcore_axis_name="s")

    @functools.partial(
        pl.kernel, mesh=mesh,
        out_type=jax.ShapeDtypeStruct((N_TOK, E_DIM), jnp.float32),
        scratch_types=[
            pltpu.VMEM((B_PER_W,), jnp.int32),
            pltpu.VMEM((B_PER_W, E_DIM), jnp.float32),
            pltpu.SemaphoreType.DMA,
        ],
    )
    def _gather(table_hbm, idx_hbm, out_hbm, idx_v, rows_v, sem):
        wid = lax.axis_index("s") * SC_NC + lax.axis_index("c")
        base = wid * B_PER_W
        pltpu.sync_copy(idx_hbm.at[pl.ds(base, B_PER_W)], idx_v)
        pltpu.async_copy(table_hbm.at[idx_v], rows_v, sem).wait()
        pltpu.sync_copy(rows_v, out_hbm.at[pl.ds(base, B_PER_W)])

    return _gather


def kernel(z, emb):
    b, c, h, w = z.shape
    zp = jnp.transpose(z, (0, 2, 3, 1))
    zf = zp.reshape(-1, c)
    szf = jnp.sum(zf ** 2, axis=1, keepdims=True)
    se2 = jnp.sum(emb ** 2, axis=1)
    idx2d, loss_sum = _dist_argmin(zf, szf, se2.reshape(1, N_E), emb)
    zq_flat = _make_gather()(emb, idx2d.reshape(N_TOK))
    z_q = zq_flat.reshape(b, h, w, c).transpose(0, 3, 1, 2)
    codebook_loss = loss_sum[0, 0] / jnp.float32(N_TOK * E_DIM)
    cls_loss = jnp.float32(0.0)
    return (z_q, codebook_loss, cls_loss, idx2d)


# speedup vs baseline: 1.1696x; 1.1252x over previous
import functools

import jax
import jax.numpy as jnp
from jax import lax
from jax.experimental import pallas as pl
from jax.experimental.pallas import tpu as pltpu
from jax.experimental.pallas import tpu_sc as plsc

N_TOK = 8192
N_E = 8192
E_DIM = 256

TT = 1024
TCODE = 4096
NI = N_TOK // TT
NJ = N_E // TCODE

SC_NC = 2
SC_NS = 16
SC_NW = SC_NC * SC_NS
B_PER_W = N_TOK // SC_NW


NJ_HALF = NJ // 2


def _dist_body(zf_ref, szf_ref, se2_ref, emb_ref, idx_ref, loss_ref,
               min1, arg1, min2, arg2, acc):
    i = pl.program_id(0)
    j = pl.program_id(1)
    mm = lax.dot_general(zf_ref[...].astype(jnp.bfloat16),
                         emb_ref[...].astype(jnp.bfloat16),
                         (((1,), (1,)), ((), ())),
                         preferred_element_type=jnp.float32)
    d = (szf_ref[...] + se2_ref[...]) - 2.0 * mm
    lmin = jnp.min(d, axis=1, keepdims=True)
    iot = lax.broadcasted_iota(jnp.int32, d.shape, 1)
    larg = jnp.min(jnp.where(d == lmin, iot, jnp.int32(2 ** 30)),
                   axis=1, keepdims=True) + j * TCODE

    @pl.when(j == 0)
    def _():
        min1[...] = lmin
        arg1[...] = larg

    @pl.when((j > 0) & (j < NJ_HALF))
    def _():
        better = lmin < min1[...]
        min1[...] = jnp.where(better, lmin, min1[...])
        arg1[...] = jnp.where(better, larg, arg1[...])

    @pl.when(j == NJ_HALF)
    def _():
        min2[...] = lmin
        arg2[...] = larg

    @pl.when(j > NJ_HALF)
    def _():
        better = lmin < min2[...]
        min2[...] = jnp.where(better, lmin, min2[...])
        arg2[...] = jnp.where(better, larg, arg2[...])

    @pl.when(j == NJ - 1)
    def _():
        v1b = min1[...].astype(jnp.bfloat16).astype(jnp.float32)
        take2 = min2[...] < v1b
        idx_ref[...] = jnp.where(take2, arg2[...], arg1[...])
        dsel = jnp.where(take2, min2[...], min1[...])

        @pl.when(i == 0)
        def _():
            acc[0, 0] = 0.0

        acc[0, 0] += jnp.sum(dsel)

        @pl.when(i == NI - 1)
        def _():
            loss_ref[0, 0] = acc[0, 0]


def _dist_argmin(zf, szf, se2r, emb):
    return pl.pallas_call(
        _dist_body,
        grid=(NI, NJ),
        in_specs=[
            pl.BlockSpec((TT, E_DIM), lambda i, j: (i, 0)),
            pl.BlockSpec((TT, 1), lambda i, j: (i, 0)),
            pl.BlockSpec((1, TCODE), lambda i, j: (0, j)),
            pl.BlockSpec((TCODE, E_DIM), lambda i, j: (j, 0)),
        ],
        out_specs=[
            pl.BlockSpec((TT, 1), lambda i, j: (i, 0)),
            pl.BlockSpec((1, 1), lambda i, j: (0, 0),
                         memory_space=pltpu.SMEM),
        ],
        out_shape=[
            jax.ShapeDtypeStruct((N_TOK, 1), jnp.int32),
            jax.ShapeDtypeStruct((1, 1), jnp.float32),
        ],
        scratch_shapes=[
            pltpu.VMEM((TT, 1), jnp.float32),
            pltpu.VMEM((TT, 1), jnp.int32),
            pltpu.VMEM((TT, 1), jnp.float32),
            pltpu.VMEM((TT, 1), jnp.int32),
            pltpu.SMEM((1, 1), jnp.float32),
        ],
    )(zf, szf, se2r, emb)


def _make_gather():
    mesh = plsc.VectorSubcoreMesh(core_axis_name="c", subcore_axis_name="s")

    @functools.partial(
        pl.kernel, mesh=mesh,
        out_type=jax.ShapeDtypeStruct((N_TOK, E_DIM), jnp.float32),
        scratch_types=[
            pltpu.VMEM((B_PER_W,), jnp.int32),
            pltpu.VMEM((B_PER_W, E_DIM), jnp.float32),
            pltpu.SemaphoreType.DMA,
        ],
    )
    def _gather(table_hbm, idx_hbm, out_hbm, idx_v, rows_v, sem):
        wid = lax.axis_index("s") * SC_NC + lax.axis_index("c")
        base = wid * B_PER_W
        pltpu.sync_copy(idx_hbm.at[pl.ds(base, B_PER_W)], idx_v)
        pltpu.async_copy(table_hbm.at[idx_v], rows_v, sem).wait()
        pltpu.sync_copy(rows_v, out_hbm.at[pl.ds(base, B_PER_W)])

    return _gather


def kernel(z, emb):
    b, c, h, w = z.shape
    zp = jnp.transpose(z, (0, 2, 3, 1))
    zf = zp.reshape(-1, c)
    szf = jnp.sum(zf ** 2, axis=1, keepdims=True)
    se2 = jnp.sum(emb ** 2, axis=1)
    idx2d, loss_sum = _dist_argmin(zf, szf, se2.reshape(1, N_E), emb)
    zq_flat = _make_gather()(emb, idx2d.reshape(N_TOK))
    z_q = zq_flat.reshape(b, h, w, c).transpose(0, 3, 1, 2)
    codebook_loss = loss_sum[0, 0] / jnp.float32(N_TOK * E_DIM)
    cls_loss = jnp.float32(0.0)
    return (z_q, codebook_loss, cls_loss, idx2d)
